# baseline (speedup 1.0000x reference)
import functools

import jax
import jax.numpy as jnp
import numpy as np
from jax.experimental import pallas as pl
from jax.experimental.pallas import tpu as pltpu

TASKS = 3
EPS = float(np.finfo(np.float64).eps)


def _moe_kernel(x_ref, wg_ref, w1_ref, b1_ref, w2_ref, b2_ref,
                out_ref, gates_ref, yacc_ref, yprev_ref, ey_ref, acc_ref, sem,
                *, n_experts, n_hc, bt):
    i = pl.program_id(0)
    e = pl.program_id(1)
    hc = pl.program_id(2)

    @pl.when(jnp.logical_and(e == 0, hc == 0))
    def _compute_gates():
        x = x_ref[...]
        for t in range(TASKS):
            logits = jax.lax.dot_general(
                x, wg_ref[t],
                (((1,), (0,)), ((), ())),
                preferred_element_type=jnp.float32)
            idx = jax.lax.broadcasted_iota(jnp.int32, logits.shape, 1)
            m1 = jnp.max(logits, axis=-1, keepdims=True)
            eq1 = logits == m1
            i1 = jnp.min(jnp.where(eq1, idx, 127), axis=-1, keepdims=True)
            first1 = idx == i1
            l2 = jnp.where(first1, -jnp.inf, logits)
            m2 = jnp.max(l2, axis=-1, keepdims=True)
            eq2 = l2 == m2
            i2 = jnp.min(jnp.where(eq2, idx, 127), axis=-1, keepdims=True)
            first2 = idx == i2
            z = jnp.exp(m2 - m1)
            g1 = 1.0 / (1.0 + z)
            g2 = z / (1.0 + z)
            gates = jnp.where(first1, g1, 0.0) + jnp.where(first2, g2, 0.0)
            gates = jnp.where(gates <= 0.0001, 0.0, gates)
            gates_ref[t] = gates

    @pl.when(e < n_experts)
    def _mlp_chunk():
        x = x_ref[...]
        w1 = w1_ref[0]
        w2 = w2_ref[0]
        h = jax.lax.dot_general(x, w1, (((1,), (1,)), ((), ())),
                                preferred_element_type=jnp.float32)
        h = jax.nn.relu(h + b1_ref[0])
        q = jax.lax.dot_general(h, w2, (((1,), (1,)), ((), ())),
                                preferred_element_type=jnp.float32)

        @pl.when(hc == 0)
        def _y_init():
            yacc_ref[...] = q

        @pl.when(jnp.logical_and(hc > 0, hc < n_hc - 1))
        def _y_acc():
            yacc_ref[...] += q

        @pl.when(hc == n_hc - 1)
        def _y_done():
            yprev_ref[...] = yacc_ref[...] + q

    ep = e - 1

    def _gate_col(t):
        g = gates_ref[t]
        eidx = jax.lax.broadcasted_iota(jnp.int32, g.shape, 1)
        return jnp.sum(jnp.where(eidx == ep, g, 0.0), axis=-1, keepdims=True)

    @pl.when(jnp.logical_and(e > 0, hc == 0))
    def _exp_prev():
        ey_ref[...] = jnp.exp(yprev_ref[...] + b2_ref[0])

    for t in range(TASKS):
        step = t + 1

        @pl.when(jnp.logical_and(e == 1, hc == step))
        def _acc_init(t=t):
            acc_ref[t] = _gate_col(t) * ey_ref[...]

        @pl.when(jnp.logical_and(jnp.logical_and(e > 1, e < n_experts), hc == step))
        def _acc_add(t=t):
            acc_ref[t] += _gate_col(t) * ey_ref[...]

        @pl.when(jnp.logical_and(e == n_experts, hc == step))
        def _acc_last(t=t):
            a = acc_ref[t] + _gate_col(t) * ey_ref[...]
            acc_ref[t] = jnp.log(jnp.where(a == 0.0, EPS, a))

    @pl.when(jnp.logical_and(e == n_experts, hc == n_hc - 1))
    def _flush():
        cp = pltpu.make_async_copy(
            acc_ref, out_ref.at[:, pl.ds(i * bt, bt), :], sem)
        cp.start()
        cp.wait()


def kernel(x, w_gate, fc1_w, fc1_b, fc2_w, fc2_b):
    B, D = x.shape
    E, H, _ = fc1_w.shape
    O = fc2_w.shape[1]
    BT = 1024
    HC = 512
    n_b = B // BT
    n_hc = H // HC
    assert n_hc >= TASKS + 1

    grid = (n_b, E + 1, n_hc)
    ew = lambda e: jnp.minimum(e, E - 1)
    ep = lambda e: jnp.clip(e - 1, 0, E - 1)
    out = pl.pallas_call(
        functools.partial(_moe_kernel, n_experts=E, n_hc=n_hc, bt=BT),
        grid=grid,
        in_specs=[
            pl.BlockSpec((BT, D), lambda i, e, hc: (i, 0)),
            pl.BlockSpec((TASKS, D, E), lambda i, e, hc: (0, 0, 0)),
            pl.BlockSpec((1, HC, D), lambda i, e, hc: (ew(e), hc, 0)),
            pl.BlockSpec((1, 1, HC), lambda i, e, hc: (ew(e), 0, hc)),
            pl.BlockSpec((1, O, HC), lambda i, e, hc: (ew(e), 0, hc)),
            pl.BlockSpec((1, 1, O), lambda i, e, hc: (ep(e), 0, 0)),
        ],
        out_specs=pl.BlockSpec(memory_space=pltpu.MemorySpace.HBM),
        out_shape=jax.ShapeDtypeStruct((TASKS, B, O), jnp.float32),
        scratch_shapes=[
            pltpu.VMEM((TASKS, BT, E), jnp.float32),
            pltpu.VMEM((BT, O), jnp.float32),
            pltpu.VMEM((BT, O), jnp.float32),
            pltpu.VMEM((BT, O), jnp.float32),
            pltpu.VMEM((TASKS, BT, O), jnp.float32),
            pltpu.SemaphoreType.DMA,
        ],
        compiler_params=pltpu.CompilerParams(
            vmem_limit_bytes=63 * 1024 * 1024),
    )(x, w_gate, fc1_w, fc1_b.reshape(E, 1, H), fc2_w, fc2_b.reshape(E, 1, O))
    return out


# speedup vs baseline: 1.0893x; 1.0893x over previous
import functools

import jax
import jax.numpy as jnp
import numpy as np
from jax.experimental import pallas as pl
from jax.experimental.pallas import tpu as pltpu

TASKS = 3
EPS = float(np.finfo(np.float64).eps)


def _moe_kernel(x_ref, wg_ref, w1_ref, b1_ref, w2_ref, b2_ref,
                out_ref, gates_ref, yprev_ref, acc_ref, sem,
                *, n_experts, bt):
    i = pl.program_id(0)
    e = pl.program_id(1)

    @pl.when(e == 0)
    def _compute_gates():
        yprev_ref[...] = jnp.zeros_like(yprev_ref)
        x = x_ref[...]
        for t in range(TASKS):
            logits = jax.lax.dot_general(
                x, wg_ref[t],
                (((1,), (0,)), ((), ())),
                preferred_element_type=jnp.float32)
            idx = jax.lax.broadcasted_iota(jnp.int32, logits.shape, 1)
            m1 = jnp.max(logits, axis=-1, keepdims=True)
            eq1 = logits == m1
            i1 = jnp.min(jnp.where(eq1, idx, 127), axis=-1, keepdims=True)
            first1 = idx == i1
            l2 = jnp.where(first1, -jnp.inf, logits)
            m2 = jnp.max(l2, axis=-1, keepdims=True)
            eq2 = l2 == m2
            i2 = jnp.min(jnp.where(eq2, idx, 127), axis=-1, keepdims=True)
            first2 = idx == i2
            z = jnp.exp(m2 - m1)
            g1 = 1.0 / (1.0 + z)
            g2 = z / (1.0 + z)
            gates = jnp.where(first1, g1, 0.0) + jnp.where(first2, g2, 0.0)
            gates = jnp.where(gates <= 0.0001, 0.0, gates)
            gates_ref[t] = gates

    ep = e - 1
    x = x_ref[...]
    w1 = w1_ref[0]
    w2 = w2_ref[0]
    h = jax.lax.dot_general(x, w1, (((1,), (1,)), ((), ())),
                            preferred_element_type=jnp.float32)
    h = jax.nn.relu(h + b1_ref[0])
    q = jax.lax.dot_general(h, w2, (((1,), (1,)), ((), ())),
                            preferred_element_type=jnp.float32)

    ey = jnp.exp(jnp.minimum(yprev_ref[...] + b2_ref[0], 80.0))
    keep = e > 0
    for t in range(TASKS):
        g = gates_ref[t]
        eidx = jax.lax.broadcasted_iota(jnp.int32, g.shape, 1)
        ge = jnp.sum(jnp.where(eidx == ep, g, 0.0), axis=-1, keepdims=True)
        acc_ref[t] = jnp.where(keep, acc_ref[t], 0.0) + ge * ey

    yprev_ref[...] = q

    @pl.when(e == n_experts)
    def _flush():
        for t in range(TASKS):
            a = acc_ref[t]
            acc_ref[t] = jnp.log(jnp.where(a == 0.0, EPS, a))
        cp = pltpu.make_async_copy(
            acc_ref, out_ref.at[:, pl.ds(i * bt, bt), :], sem)
        cp.start()
        cp.wait()


def kernel(x, w_gate, fc1_w, fc1_b, fc2_w, fc2_b):
    B, D = x.shape
    E, H, _ = fc1_w.shape
    O = fc2_w.shape[1]
    BT = 512
    n_b = B // BT

    grid = (n_b, E + 1)
    ew = lambda e: jnp.minimum(e, E - 1)
    epc = lambda e: jnp.clip(e - 1, 0, E - 1)
    out = pl.pallas_call(
        functools.partial(_moe_kernel, n_experts=E, bt=BT),
        grid=grid,
        in_specs=[
            pl.BlockSpec((BT, D), lambda i, e: (i, 0)),
            pl.BlockSpec((TASKS, D, E), lambda i, e: (0, 0, 0)),
            pl.BlockSpec((1, H, D), lambda i, e: (ew(e), 0, 0)),
            pl.BlockSpec((1, 1, H), lambda i, e: (ew(e), 0, 0)),
            pl.BlockSpec((1, O, H), lambda i, e: (ew(e), 0, 0)),
            pl.BlockSpec((1, 1, O), lambda i, e: (epc(e), 0, 0)),
        ],
        out_specs=pl.BlockSpec(memory_space=pltpu.MemorySpace.HBM),
        out_shape=jax.ShapeDtypeStruct((TASKS, B, O), jnp.float32),
        scratch_shapes=[
            pltpu.VMEM((TASKS, BT, E), jnp.float32),
            pltpu.VMEM((BT, O), jnp.float32),
            pltpu.VMEM((TASKS, BT, O), jnp.float32),
            pltpu.SemaphoreType.DMA,
        ],
        compiler_params=pltpu.CompilerParams(
            vmem_limit_bytes=63 * 1024 * 1024),
    )(x, w_gate, fc1_w, fc1_b.reshape(E, 1, H), fc2_w, fc2_b.reshape(E, 1, O))
    return out


# speedup vs baseline: 1.0940x; 1.0044x over previous
import functools

import jax
import jax.numpy as jnp
import numpy as np
from jax.experimental import pallas as pl
from jax.experimental.pallas import tpu as pltpu

TASKS = 3
EPS = float(np.finfo(np.float64).eps)


def _gating(x, wg_ref, g_ref):
    for t in range(TASKS):
        logits = jax.lax.dot_general(
            x, wg_ref[t],
            (((1,), (0,)), ((), ())),
            preferred_element_type=jnp.float32)
        idx = jax.lax.broadcasted_iota(jnp.int32, logits.shape, 1)
        m1 = jnp.max(logits, axis=-1, keepdims=True)
        eq1 = logits == m1
        i1 = jnp.min(jnp.where(eq1, idx, 127), axis=-1, keepdims=True)
        first1 = idx == i1
        l2 = jnp.where(first1, -jnp.inf, logits)
        m2 = jnp.max(l2, axis=-1, keepdims=True)
        eq2 = l2 == m2
        i2 = jnp.min(jnp.where(eq2, idx, 127), axis=-1, keepdims=True)
        first2 = idx == i2
        z = jnp.exp(m2 - m1)
        g1 = 1.0 / (1.0 + z)
        g2 = z / (1.0 + z)
        gates = jnp.where(first1, g1, 0.0) + jnp.where(first2, g2, 0.0)
        gates = jnp.where(gates <= 0.0001, 0.0, gates)
        g_ref[t] = gates


def _moe_kernel(x_ref, wg_ref, w1_ref, b1_ref, w2_ref, b2_ref,
                out_ref, g0_ref, g1_ref, yprev_ref, acc_ref, sem,
                *, n_experts, n_b, bt):
    s = pl.program_id(0)
    e = jax.lax.rem(s, n_experts)
    i = jax.lax.div(s, n_experts)
    sp = jnp.maximum(s - 1, 0)
    ep = jax.lax.rem(sp, n_experts)
    ip = jax.lax.div(sp, n_experts)
    valid = s > 0

    ey = jnp.exp(jnp.minimum(yprev_ref[...] + b2_ref[0], 80.0))
    cpar = jax.lax.rem(ip, 2)
    keep = jnp.logical_and(valid, ep > 0)
    for t in range(TASKS):
        ga, gb = g0_ref[t], g1_ref[t]
        eidx = jax.lax.broadcasted_iota(jnp.int32, ga.shape, 1)
        sel = jnp.where(cpar == 0, ga, gb)
        ge = jnp.sum(jnp.where(eidx == ep, sel, 0.0), axis=-1, keepdims=True)
        contrib = jnp.where(valid, ge * ey, 0.0)
        acc_ref[t] = jnp.where(keep, acc_ref[t], 0.0) + contrib

    @pl.when(jnp.logical_and(valid, ep == n_experts - 1))
    def _flush():
        for t in range(TASKS):
            a = acc_ref[t]
            acc_ref[t] = jnp.log(jnp.where(a == 0.0, EPS, a))
        cp = pltpu.make_async_copy(
            acc_ref, out_ref.at[:, pl.ds(ip * bt, bt), :], sem)
        cp.start()
        cp.wait()

    @pl.when(jnp.logical_and(e == 0, s < n_b * n_experts))
    def _compute_gates():
        @pl.when(s == 0)
        def _init():
            yprev_ref[...] = jnp.zeros_like(yprev_ref)

        x = x_ref[...]

        @pl.when(jax.lax.rem(i, 2) == 0)
        def _even():
            _gating(x, wg_ref, g0_ref)

        @pl.when(jax.lax.rem(i, 2) == 1)
        def _odd():
            _gating(x, wg_ref, g1_ref)

    x = x_ref[...]
    w1 = w1_ref[0]
    w2 = w2_ref[0]
    h = jax.lax.dot_general(x, w1, (((1,), (1,)), ((), ())),
                            preferred_element_type=jnp.float32)
    h = jax.nn.relu(h + b1_ref[0])
    q = jax.lax.dot_general(h, w2, (((1,), (1,)), ((), ())),
                            preferred_element_type=jnp.float32)
    yprev_ref[...] = q


def kernel(x, w_gate, fc1_w, fc1_b, fc2_w, fc2_b):
    B, D = x.shape
    E, H, _ = fc1_w.shape
    O = fc2_w.shape[1]
    BT = 512
    n_b = B // BT

    grid = (n_b * E + 1,)
    out = pl.pallas_call(
        functools.partial(_moe_kernel, n_experts=E, n_b=n_b, bt=BT),
        grid=grid,
        in_specs=[
            pl.BlockSpec((BT, D),
                         lambda s: (jnp.minimum(jax.lax.div(s, E), n_b - 1), 0)),
            pl.BlockSpec((TASKS, D, E), lambda s: (0, 0, 0)),
            pl.BlockSpec((1, H, D), lambda s: (jax.lax.rem(s, E), 0, 0)),
            pl.BlockSpec((1, 1, H), lambda s: (jax.lax.rem(s, E), 0, 0)),
            pl.BlockSpec((1, O, H), lambda s: (jax.lax.rem(s, E), 0, 0)),
            pl.BlockSpec((1, 1, O), lambda s: (jnp.mod(s - 1, E), 0, 0)),
        ],
        out_specs=pl.BlockSpec(memory_space=pltpu.MemorySpace.HBM),
        out_shape=jax.ShapeDtypeStruct((TASKS, B, O), jnp.float32),
        scratch_shapes=[
            pltpu.VMEM((TASKS, BT, E), jnp.float32),
            pltpu.VMEM((TASKS, BT, E), jnp.float32),
            pltpu.VMEM((BT, O), jnp.float32),
            pltpu.VMEM((TASKS, BT, O), jnp.float32),
            pltpu.SemaphoreType.DMA,
        ],
        compiler_params=pltpu.CompilerParams(
            vmem_limit_bytes=63 * 1024 * 1024),
    )(x, w_gate, fc1_w, fc1_b.reshape(E, 1, H), fc2_w, fc2_b.reshape(E, 1, O))
    return out


# speedup vs baseline: 1.1313x; 1.0341x over previous
import functools

import jax
import jax.numpy as jnp
import numpy as np
from jax.experimental import pallas as pl
from jax.experimental.pallas import tpu as pltpu

TASKS = 3
EPS = float(np.finfo(np.float64).eps)


def _moe_kernel(x_ref, wg_ref, w1_ref, b1_ref, w2_ref, b2_ref,
                out_ref, gates_ref, h_ref, acc_ref, sem,
                *, n_experts, n_oc, bt, oc_sz):
    i = pl.program_id(0)
    e = pl.program_id(1)
    oc = pl.program_id(2)

    @pl.when(jnp.logical_and(e == 0, oc == 0))
    def _compute_gates():
        x = x_ref[...]
        for t in range(TASKS):
            logits = jax.lax.dot_general(
                x, wg_ref[t],
                (((1,), (0,)), ((), ())),
                preferred_element_type=jnp.float32)
            idx = jax.lax.broadcasted_iota(jnp.int32, logits.shape, 1)
            m1 = jnp.max(logits, axis=-1, keepdims=True)
            eq1 = logits == m1
            i1 = jnp.min(jnp.where(eq1, idx, 127), axis=-1, keepdims=True)
            first1 = idx == i1
            l2 = jnp.where(first1, -jnp.inf, logits)
            m2 = jnp.max(l2, axis=-1, keepdims=True)
            eq2 = l2 == m2
            i2 = jnp.min(jnp.where(eq2, idx, 127), axis=-1, keepdims=True)
            first2 = idx == i2
            z = jnp.exp(m2 - m1)
            g1 = 1.0 / (1.0 + z)
            g2 = z / (1.0 + z)
            gates = jnp.where(first1, g1, 0.0) + jnp.where(first2, g2, 0.0)
            gates = jnp.where(gates <= 0.0001, 0.0, gates)
            gates_ref[t] = gates

    @pl.when(oc == 0)
    def _fc1():
        x = x_ref[...]
        w1 = w1_ref[0]
        h = jax.lax.dot_general(x, w1, (((1,), (1,)), ((), ())),
                                preferred_element_type=jnp.float32)
        h_ref[...] = jax.nn.relu(h + b1_ref[0])

    w2 = w2_ref[0]
    y = jax.lax.dot_general(h_ref[...], w2, (((1,), (1,)), ((), ())),
                            preferred_element_type=jnp.float32)
    ey = jnp.exp(y + b2_ref[0])
    for t in range(TASKS):
        g = gates_ref[t]
        eidx = jax.lax.broadcasted_iota(jnp.int32, g.shape, 1)
        ge = jnp.sum(jnp.where(eidx == e, g, 0.0), axis=-1, keepdims=True)
        acc_ref[t, oc] = (jnp.where(e > 0, acc_ref[t, oc], 0.0) + ge * ey)

    @pl.when(e == n_experts - 1)
    def _flush():
        for t in range(TASKS):
            a = acc_ref[t, oc]
            acc_ref[t, oc] = jnp.log(jnp.where(a == 0.0, EPS, a))
        cp = pltpu.make_async_copy(
            acc_ref.at[:, oc],
            out_ref.at[:, pl.ds(i * bt, bt), pl.ds(oc * oc_sz, oc_sz)],
            sem)
        cp.start()
        cp.wait()


def kernel(x, w_gate, fc1_w, fc1_b, fc2_w, fc2_b):
    B, D = x.shape
    E, H, _ = fc1_w.shape
    O = fc2_w.shape[1]
    BT = 1024
    OC = 512
    n_b = B // BT
    n_oc = O // OC

    grid = (n_b, E, n_oc)
    out = pl.pallas_call(
        functools.partial(_moe_kernel, n_experts=E, n_oc=n_oc, bt=BT, oc_sz=OC),
        grid=grid,
        in_specs=[
            pl.BlockSpec((BT, D), lambda i, e, oc: (i, 0)),
            pl.BlockSpec((TASKS, D, E), lambda i, e, oc: (0, 0, 0)),
            pl.BlockSpec((1, H, D), lambda i, e, oc: (e, 0, 0)),
            pl.BlockSpec((1, 1, H), lambda i, e, oc: (e, 0, 0)),
            pl.BlockSpec((1, OC, H), lambda i, e, oc: (e, oc, 0)),
            pl.BlockSpec((1, 1, OC), lambda i, e, oc: (e, 0, oc)),
        ],
        out_specs=pl.BlockSpec(memory_space=pltpu.MemorySpace.HBM),
        out_shape=jax.ShapeDtypeStruct((TASKS, B, O), jnp.float32),
        scratch_shapes=[
            pltpu.VMEM((TASKS, BT, E), jnp.float32),
            pltpu.VMEM((BT, H), jnp.float32),
            pltpu.VMEM((TASKS, O // OC, BT, OC), jnp.float32),
            pltpu.SemaphoreType.DMA,
        ],
        compiler_params=pltpu.CompilerParams(
            vmem_limit_bytes=63 * 1024 * 1024),
    )(x, w_gate, fc1_w, fc1_b.reshape(E, 1, H), fc2_w, fc2_b.reshape(E, 1, O))
    return out


# speedup vs baseline: 1.1971x; 1.0581x over previous
import functools

import jax
import jax.numpy as jnp
import numpy as np
from jax.experimental import pallas as pl
from jax.experimental.pallas import tpu as pltpu

TASKS = 3
EPS = float(np.finfo(np.float64).eps)


def _moe_kernel(x_ref, wg_ref, w1_ref, b1_ref, w2_ref, b2_ref,
                out_ref, gates_ref, *, n_experts):
    e = pl.program_id(1)

    @pl.when(e == 0)
    def _compute_gates():
        x = x_ref[...]
        for t in range(TASKS):
            logits = jax.lax.dot_general(
                x, wg_ref[t],
                (((1,), (0,)), ((), ())),
                preferred_element_type=jnp.float32)
            idx = jax.lax.broadcasted_iota(jnp.int32, logits.shape, 1)
            m1 = jnp.max(logits, axis=-1, keepdims=True)
            eq1 = logits == m1
            i1 = jnp.min(jnp.where(eq1, idx, 127), axis=-1, keepdims=True)
            first1 = idx == i1
            l2 = jnp.where(first1, -jnp.inf, logits)
            m2 = jnp.max(l2, axis=-1, keepdims=True)
            eq2 = l2 == m2
            i2 = jnp.min(jnp.where(eq2, idx, 127), axis=-1, keepdims=True)
            first2 = idx == i2
            z = jnp.exp(m2 - m1)
            g1 = 1.0 / (1.0 + z)
            g2 = z / (1.0 + z)
            gates = jnp.where(first1, g1, 0.0) + jnp.where(first2, g2, 0.0)
            gates = jnp.where(gates <= 0.0001, 0.0, gates)
            gates_ref[t] = gates

    x = x_ref[...]
    w1 = w1_ref[0]
    w2 = w2_ref[0]
    h = jax.lax.dot_general(x, w1, (((1,), (1,)), ((), ())),
                            preferred_element_type=jnp.float32)
    h = jax.nn.relu(h + b1_ref[0])
    y = jax.lax.dot_general(h, w2, (((1,), (1,)), ((), ())),
                            preferred_element_type=jnp.float32)
    y = y + b2_ref[0]
    ey = jnp.exp(y)

    gates_all = gates_ref[...]
    eidx = jax.lax.broadcasted_iota(jnp.int32, gates_all.shape, 2)
    ge = jnp.sum(jnp.where(eidx == e, gates_all, 0.0), axis=-1)
    contrib = ge[:, :, None] * ey[None, :, :]

    @pl.when(e == 0)
    def _init():
        out_ref[...] = contrib

    @pl.when(e > 0)
    def _acc():
        out_ref[...] += contrib

    @pl.when(e == n_experts - 1)
    def _finish():
        acc = out_ref[...]
        out_ref[...] = jnp.log(jnp.where(acc == 0.0, EPS, acc))


def kernel(x, w_gate, fc1_w, fc1_b, fc2_w, fc2_b):
    B, D = x.shape
    E, H, _ = fc1_w.shape
    O = fc2_w.shape[1]
    BT = 512
    n_b = B // BT

    grid = (n_b, E)
    out = pl.pallas_call(
        functools.partial(_moe_kernel, n_experts=E),
        grid=grid,
        in_specs=[
            pl.BlockSpec((BT, D), lambda i, e: (i, 0)),
            pl.BlockSpec((TASKS, D, E), lambda i, e: (0, 0, 0)),
            pl.BlockSpec((1, H, D), lambda i, e: (e, 0, 0)),
            pl.BlockSpec((1, 1, H), lambda i, e: (e, 0, 0)),
            pl.BlockSpec((1, O, H), lambda i, e: (e, 0, 0)),
            pl.BlockSpec((1, 1, O), lambda i, e: (e, 0, 0)),
        ],
        out_specs=pl.BlockSpec((TASKS, BT, O), lambda i, e: (0, i, 0)),
        out_shape=jax.ShapeDtypeStruct((TASKS, B, O), jnp.float32),
        scratch_shapes=[pltpu.VMEM((TASKS, BT, E), jnp.float32)],
        compiler_params=pltpu.CompilerParams(
            vmem_limit_bytes=63 * 1024 * 1024),
    )(x, w_gate, fc1_w, fc1_b.reshape(E, 1, H), fc2_w, fc2_b.reshape(E, 1, O))
    return out
